# baseline (speedup 1.0000x reference)
import functools

import jax
import jax.numpy as jnp
from jax import lax
from jax.experimental import pallas as pl
from jax.experimental.pallas import tpu as pltpu
from jax.experimental.pallas import tpu_sc as plsc

N = 10000
D = 128
E = 320000

NC = 2
NS = 16
NPAD = 10240
ROWS_PER_TILE = NPAD // NS
CHUNK = 80
CPT = 128
E_PAD = NC * NS * CPT * CHUNK
EROWS = E_PAD // CHUNK
N_PADROWS = NPAD - N
R = 4

_MESH = dict(core_axis_name="c", subcore_axis_name="s")


def _sc_deg(dst2d):
    @functools.partial(
        pl.kernel,
        mesh=plsc.VectorSubcoreMesh(**_MESH),
        out_type=jax.ShapeDtypeStruct((NC * NPAD, D), jnp.float32),
        scratch_types=[
            pltpu.VMEM_SHARED((NPAD, D), jnp.float32),
            pltpu.VMEM((CPT, CHUNK), jnp.int32),
            pltpu.VMEM((CHUNK, D), jnp.float32),
            pltpu.SemaphoreType.DMA,
        ],
    )
    def k(dst_hbm, out_hbm, acc, didx, ones_v, sem):
        c = lax.axis_index("c")
        s = lax.axis_index("s")
        wid = c * NS + s
        r0 = s * ROWS_PER_TILE
        pltpu.sync_copy(dst_hbm.at[pl.ds(wid * CPT, CPT)], didx)

        zv = jnp.zeros((16,), jnp.float32)
        e0 = jnp.where(lax.iota(jnp.int32, 16) == 0, 1.0, 0.0).astype(jnp.float32)

        def zrow(r, carry):
            for l in range(D // 16):
                ones_v[r, pl.ds(l * 16, 16)] = zv
            return carry

        lax.fori_loop(0, CHUNK, zrow, 0)
        for b in range(ROWS_PER_TILE // CHUNK):
            pltpu.sync_copy(ones_v, acc.at[pl.ds(r0 + b * CHUNK, CHUNK)])

        def orow(r, carry):
            ones_v[r, pl.ds(0, 16)] = e0
            return carry

        lax.fori_loop(0, CHUNK, orow, 0)
        plsc.subcore_barrier()

        W = 8
        for j in range(W):
            pltpu.async_copy(ones_v, acc.at[didx.at[j]], sem, add=True)

        def body(j, carry):
            pltpu.make_async_copy(ones_v, acc.at[didx.at[j]], sem).wait()
            pltpu.async_copy(ones_v, acc.at[didx.at[j + W]], sem, add=True)
            return carry

        lax.fori_loop(0, CPT - W, body, 0)
        for j in range(CPT - W, CPT):
            pltpu.make_async_copy(ones_v, acc.at[didx.at[j]], sem).wait()
        plsc.subcore_barrier()
        pltpu.sync_copy(acc.at[pl.ds(r0, ROWS_PER_TILE)],
                        out_hbm.at[pl.ds(c * NPAD + r0, ROWS_PER_TILE)])

    return k(dst2d)


def _sc_agg(hp, src2d, dst2d):
    @functools.partial(
        pl.kernel,
        mesh=plsc.VectorSubcoreMesh(**_MESH),
        out_type=jax.ShapeDtypeStruct((NC * NPAD, D), jnp.float32),
        scratch_types=[
            pltpu.VMEM_SHARED((NPAD, D), jnp.float32),
            pltpu.VMEM((R, CHUNK), jnp.int32),
            pltpu.VMEM((R, CHUNK), jnp.int32),
        ] + [pltpu.VMEM((CHUNK, D), jnp.float32)] * R
          + [pltpu.SemaphoreType.DMA] * (4 * R),
    )
    def k(hp_hbm, src_hbm, dst_hbm, out_hbm, acc, si, di, *rest):
        DB = list(rest[:R])
        gsem = list(rest[R:2 * R])
        ssem = list(rest[2 * R:3 * R])
        issem = list(rest[3 * R:4 * R])
        idsem = list(rest[4 * R:5 * R])
        c = lax.axis_index("c")
        s = lax.axis_index("s")
        wid = c * NS + s
        r0 = s * ROWS_PER_TILE
        ch0 = wid * CPT
        pltpu.sync_copy(hp_hbm.at[pl.ds(r0, ROWS_PER_TILE)],
                        acc.at[pl.ds(r0, ROWS_PER_TILE)])
        plsc.subcore_barrier()

        def load_si(g, m):
            pltpu.async_copy(src_hbm.at[ch0 + g], si.at[m], issem[m])

        def wait_si(m):
            pltpu.make_async_copy(src_hbm.at[ch0], si.at[m], issem[m]).wait()

        def load_di(g, m):
            pltpu.async_copy(dst_hbm.at[ch0 + g], di.at[m], idsem[m])

        def wait_di(m):
            pltpu.make_async_copy(dst_hbm.at[ch0], di.at[m], idsem[m]).wait()

        def start_g(m):
            pltpu.async_copy(hp_hbm.at[si.at[m]], DB[m], gsem[m])

        def wait_g(m):
            pltpu.make_async_copy(hp_hbm.at[si.at[m]], DB[m], gsem[m]).wait()

        def start_s(m):
            pltpu.async_copy(DB[m], acc.at[di.at[m]], ssem[m], add=True)

        def wait_s(m):
            pltpu.make_async_copy(DB[m], acc.at[di.at[0]], ssem[m]).wait()

        def emit(g, m, first=False, dwait=True, lsrc=True, ldst=True,
                 nxt=True):
            mp = (m + R - 1) % R
            wait_g(m)
            if dwait:
                wait_di(m)
            start_s(m)
            if not first:
                wait_s(mp)
            if lsrc:
                load_si(g + R, m)
            if ldst:
                load_di(g + R - 1, mp)
            if nxt:
                wait_si(mp)
                start_g(mp)

        for m in range(R - 1):
            pltpu.sync_copy(src_hbm.at[ch0 + m], si.at[m])
            pltpu.sync_copy(dst_hbm.at[ch0 + m], di.at[m])
        for m in range(R - 1):
            start_g(m)
        load_si(R - 1, R - 1)
        for g in range(R - 1):
            emit(g, g, first=(g == 0), dwait=False)

        def body(rr, carry):
            gg = (R - 1) + R * rr
            for u in range(R):
                emit(gg + u, (R - 1 + u) % R)
            return carry

        nsteady = (CPT - (R - 1) - (R + 1)) // R
        lax.fori_loop(0, nsteady, body, 0)
        for g in range(R - 1 + nsteady * R, CPT):
            emit(g, g % R, lsrc=(g + R < CPT), ldst=(g + R - 1 < CPT),
                 nxt=(g + R - 1 < CPT))
        wait_s((CPT - 1) % R)
        plsc.subcore_barrier()
        pltpu.sync_copy(acc.at[pl.ds(r0, ROWS_PER_TILE)],
                        out_hbm.at[pl.ds(c * NPAD + r0, ROWS_PER_TILE)])

    return k(hp, src2d, dst2d)


_BR = 1024


def _tc_mm_dis(xp, W, degp):
    def body(x_ref, w_ref, p_ref, hp_ref, dis_ref):
        i = pl.program_id(0)
        g1 = lax.dot_general(
            x_ref[...], w_ref[...], (((1,), (0,)), ((), ())),
            precision=lax.Precision.HIGHEST,
            preferred_element_type=jnp.float32)
        degc = p_ref[0, :, 0:1] + p_ref[1, :, 0:1] + 1.0
        dis = lax.rsqrt(degc)
        rows = i * _BR + lax.broadcasted_iota(jnp.int32, (_BR, 1), 0)
        dis = jnp.where(rows < N, dis, 0.0)
        disb = jnp.broadcast_to(dis, (_BR, D))
        dis_ref[...] = disb
        hp_ref[...] = g1 * disb

    return pl.pallas_call(
        body,
        grid=(NPAD // _BR,),
        in_specs=[pl.BlockSpec((_BR, D), lambda i: (i, 0)),
                  pl.BlockSpec((D, D), lambda i: (0, 0)),
                  pl.BlockSpec((NC, _BR, D), lambda i: (0, i, 0))],
        out_specs=[pl.BlockSpec((_BR, D), lambda i: (i, 0)),
                   pl.BlockSpec((_BR, D), lambda i: (i, 0))],
        out_shape=[jax.ShapeDtypeStruct((NPAD, D), jnp.float32),
                   jax.ShapeDtypeStruct((NPAD, D), jnp.float32)],
    )(xp, W, degp.reshape(NC, NPAD, D))


def _tc_mid(parts, hp1, disf, b1, W2):
    def body(p_ref, hp_ref, dis_ref, b_ref, w_ref, o_ref):
        x1 = dis_ref[...] * (p_ref[0] + p_ref[1] - hp_ref[...]) + b_ref[...]
        t = jnp.maximum(x1, 0.0)
        g2 = lax.dot_general(t, w_ref[...], (((1,), (0,)), ((), ())),
                             precision=lax.Precision.HIGHEST,
                             preferred_element_type=jnp.float32)
        o_ref[...] = dis_ref[...] * g2

    return pl.pallas_call(
        body,
        grid=(NPAD // _BR,),
        in_specs=[pl.BlockSpec((NC, _BR, D), lambda i: (0, i, 0)),
                  pl.BlockSpec((_BR, D), lambda i: (i, 0)),
                  pl.BlockSpec((_BR, D), lambda i: (i, 0)),
                  pl.BlockSpec((1, D), lambda i: (0, 0)),
                  pl.BlockSpec((D, D), lambda i: (0, 0))],
        out_specs=pl.BlockSpec((_BR, D), lambda i: (i, 0)),
        out_shape=jax.ShapeDtypeStruct((NPAD, D), jnp.float32),
    )(parts.reshape(NC, NPAD, D), hp1, disf, b1.reshape(1, D), W2)


def _tc_final(parts, hp2, disf, b2):
    def body(p_ref, hp_ref, dis_ref, b_ref, o_ref):
        o = dis_ref[...] * (p_ref[0] + p_ref[1] - hp_ref[...]) + b_ref[...]
        m = jnp.max(o, axis=1, keepdims=True)
        e = jnp.exp(o - m)
        lse = jnp.log(jnp.sum(e, axis=1, keepdims=True))
        o_ref[...] = o - m - lse

    return pl.pallas_call(
        body,
        grid=(NPAD // _BR,),
        in_specs=[pl.BlockSpec((NC, _BR, D), lambda i: (0, i, 0)),
                  pl.BlockSpec((_BR, D), lambda i: (i, 0)),
                  pl.BlockSpec((_BR, D), lambda i: (i, 0)),
                  pl.BlockSpec((1, D), lambda i: (0, 0))],
        out_specs=pl.BlockSpec((_BR, D), lambda i: (i, 0)),
        out_shape=jax.ShapeDtypeStruct((NPAD, D), jnp.float32),
    )(parts.reshape(NC, NPAD, D), hp2, disf, b2.reshape(1, D))


def kernel(x, edge_index, W1, b1, W2, b2):
    ei = edge_index.astype(jnp.int32)
    npe = E_PAD - E
    padv = N + (jnp.arange(npe, dtype=jnp.int32) % N_PADROWS)
    src2d = jnp.concatenate([ei[0], padv]).reshape(EROWS, CHUNK)
    dst2d = jnp.concatenate([ei[1], padv]).reshape(EROWS, CHUNK)
    xp = jnp.pad(x, ((0, NPAD - N), (0, 0)))

    degp = _sc_deg(dst2d)
    hp1, disf = _tc_mm_dis(xp, W1, degp)

    parts1 = _sc_agg(hp1, src2d, dst2d)
    hp2 = _tc_mid(parts1, hp1, disf, b1, W2)

    parts2 = _sc_agg(hp2, src2d, dst2d)
    out = _tc_final(parts2, hp2, disf, b2)
    return out[:N]


# speedup vs baseline: 1.0049x; 1.0049x over previous
import functools

import jax
import jax.numpy as jnp
from jax import lax
from jax.experimental import pallas as pl
from jax.experimental.pallas import tpu as pltpu
from jax.experimental.pallas import tpu_sc as plsc

N = 10000
D = 128
E = 320000

NC = 2
NS = 16
NPAD = 10240
ROWS_PER_TILE = NPAD // NS
CHUNK = 64
CPT = 160
E_PAD = NC * NS * CPT * CHUNK
EROWS = E_PAD // CHUNK
N_PADROWS = NPAD - N
R = 5

_MESH = dict(core_axis_name="c", subcore_axis_name="s")


def _sc_deg(dst2d):
    @functools.partial(
        pl.kernel,
        mesh=plsc.VectorSubcoreMesh(**_MESH),
        out_type=jax.ShapeDtypeStruct((NC * NPAD, D), jnp.float32),
        scratch_types=[
            pltpu.VMEM_SHARED((NPAD, D), jnp.float32),
            pltpu.VMEM((CPT, CHUNK), jnp.int32),
            pltpu.VMEM((CHUNK, D), jnp.float32),
            pltpu.SemaphoreType.DMA,
        ],
    )
    def k(dst_hbm, out_hbm, acc, didx, ones_v, sem):
        c = lax.axis_index("c")
        s = lax.axis_index("s")
        wid = c * NS + s
        r0 = s * ROWS_PER_TILE
        pltpu.sync_copy(dst_hbm.at[pl.ds(wid * CPT, CPT)], didx)

        zv = jnp.zeros((16,), jnp.float32)
        e0 = jnp.where(lax.iota(jnp.int32, 16) == 0, 1.0, 0.0).astype(jnp.float32)

        def zrow(r, carry):
            for l in range(D // 16):
                ones_v[r, pl.ds(l * 16, 16)] = zv
            return carry

        lax.fori_loop(0, CHUNK, zrow, 0)
        for b in range(ROWS_PER_TILE // CHUNK):
            pltpu.sync_copy(ones_v, acc.at[pl.ds(r0 + b * CHUNK, CHUNK)])

        def orow(r, carry):
            ones_v[r, pl.ds(0, 16)] = e0
            return carry

        lax.fori_loop(0, CHUNK, orow, 0)
        plsc.subcore_barrier()

        W = 8
        for j in range(W):
            pltpu.async_copy(ones_v, acc.at[didx.at[j]], sem, add=True)

        def body(j, carry):
            pltpu.make_async_copy(ones_v, acc.at[didx.at[j]], sem).wait()
            pltpu.async_copy(ones_v, acc.at[didx.at[j + W]], sem, add=True)
            return carry

        lax.fori_loop(0, CPT - W, body, 0)
        for j in range(CPT - W, CPT):
            pltpu.make_async_copy(ones_v, acc.at[didx.at[j]], sem).wait()
        plsc.subcore_barrier()
        pltpu.sync_copy(acc.at[pl.ds(r0, ROWS_PER_TILE)],
                        out_hbm.at[pl.ds(c * NPAD + r0, ROWS_PER_TILE)])

    return k(dst2d)


def _sc_agg(hp, src2d, dst2d):
    @functools.partial(
        pl.kernel,
        mesh=plsc.VectorSubcoreMesh(**_MESH),
        out_type=jax.ShapeDtypeStruct((NC * NPAD, D), jnp.float32),
        scratch_types=[
            pltpu.VMEM_SHARED((NPAD, D), jnp.float32),
            pltpu.VMEM((R, CHUNK), jnp.int32),
            pltpu.VMEM((R, CHUNK), jnp.int32),
        ] + [pltpu.VMEM((CHUNK, D), jnp.float32)] * R
          + [pltpu.SemaphoreType.DMA] * (4 * R),
    )
    def k(hp_hbm, src_hbm, dst_hbm, out_hbm, acc, si, di, *rest):
        DB = list(rest[:R])
        gsem = list(rest[R:2 * R])
        ssem = list(rest[2 * R:3 * R])
        issem = list(rest[3 * R:4 * R])
        idsem = list(rest[4 * R:5 * R])
        c = lax.axis_index("c")
        s = lax.axis_index("s")
        wid = c * NS + s
        r0 = s * ROWS_PER_TILE
        ch0 = wid * CPT
        pltpu.sync_copy(hp_hbm.at[pl.ds(r0, ROWS_PER_TILE)],
                        acc.at[pl.ds(r0, ROWS_PER_TILE)])
        plsc.subcore_barrier()

        def load_si(g, m):
            pltpu.async_copy(src_hbm.at[ch0 + g], si.at[m], issem[m])

        def wait_si(m):
            pltpu.make_async_copy(src_hbm.at[ch0], si.at[m], issem[m]).wait()

        def load_di(g, m):
            pltpu.async_copy(dst_hbm.at[ch0 + g], di.at[m], idsem[m])

        def wait_di(m):
            pltpu.make_async_copy(dst_hbm.at[ch0], di.at[m], idsem[m]).wait()

        def start_g(m):
            pltpu.async_copy(hp_hbm.at[si.at[m]], DB[m], gsem[m])

        def wait_g(m):
            pltpu.make_async_copy(hp_hbm.at[si.at[m]], DB[m], gsem[m]).wait()

        def start_s(m):
            pltpu.async_copy(DB[m], acc.at[di.at[m]], ssem[m], add=True)

        def wait_s(m):
            pltpu.make_async_copy(DB[m], acc.at[di.at[0]], ssem[m]).wait()

        def emit(g, m, first=False, dwait=True, lsrc=True, ldst=True,
                 nxt=True):
            mp = (m + R - 1) % R
            wait_g(m)
            if dwait:
                wait_di(m)
            start_s(m)
            if not first:
                wait_s(mp)
            if lsrc:
                load_si(g + R, m)
            if ldst:
                load_di(g + R - 1, mp)
            if nxt:
                wait_si(mp)
                start_g(mp)

        for m in range(R - 1):
            pltpu.sync_copy(src_hbm.at[ch0 + m], si.at[m])
            pltpu.sync_copy(dst_hbm.at[ch0 + m], di.at[m])
        for m in range(R - 1):
            start_g(m)
        load_si(R - 1, R - 1)
        for g in range(R - 1):
            emit(g, g, first=(g == 0), dwait=False)

        def body(rr, carry):
            gg = (R - 1) + R * rr
            for u in range(R):
                emit(gg + u, (R - 1 + u) % R)
            return carry

        nsteady = (CPT - (R - 1) - (R + 1)) // R
        lax.fori_loop(0, nsteady, body, 0)
        for g in range(R - 1 + nsteady * R, CPT):
            emit(g, g % R, lsrc=(g + R < CPT), ldst=(g + R - 1 < CPT),
                 nxt=(g + R - 1 < CPT))
        wait_s((CPT - 1) % R)
        plsc.subcore_barrier()
        pltpu.sync_copy(acc.at[pl.ds(r0, ROWS_PER_TILE)],
                        out_hbm.at[pl.ds(c * NPAD + r0, ROWS_PER_TILE)])

    return k(hp, src2d, dst2d)


_BR = 1024


def _tc_mm_dis(xp, W, degp):
    def body(x_ref, w_ref, p_ref, hp_ref, dis_ref):
        i = pl.program_id(0)
        g1 = lax.dot_general(
            x_ref[...], w_ref[...], (((1,), (0,)), ((), ())),
            precision=lax.Precision.HIGHEST,
            preferred_element_type=jnp.float32)
        degc = p_ref[0, :, 0:1] + p_ref[1, :, 0:1] + 1.0
        dis = lax.rsqrt(degc)
        rows = i * _BR + lax.broadcasted_iota(jnp.int32, (_BR, 1), 0)
        dis = jnp.where(rows < N, dis, 0.0)
        disb = jnp.broadcast_to(dis, (_BR, D))
        dis_ref[...] = disb
        hp_ref[...] = g1 * disb

    return pl.pallas_call(
        body,
        grid=(NPAD // _BR,),
        in_specs=[pl.BlockSpec((_BR, D), lambda i: (i, 0)),
                  pl.BlockSpec((D, D), lambda i: (0, 0)),
                  pl.BlockSpec((NC, _BR, D), lambda i: (0, i, 0))],
        out_specs=[pl.BlockSpec((_BR, D), lambda i: (i, 0)),
                   pl.BlockSpec((_BR, D), lambda i: (i, 0))],
        out_shape=[jax.ShapeDtypeStruct((NPAD, D), jnp.float32),
                   jax.ShapeDtypeStruct((NPAD, D), jnp.float32)],
    )(xp, W, degp.reshape(NC, NPAD, D))


def _tc_mid(parts, hp1, disf, b1, W2):
    def body(p_ref, hp_ref, dis_ref, b_ref, w_ref, o_ref):
        x1 = dis_ref[...] * (p_ref[0] + p_ref[1] - hp_ref[...]) + b_ref[...]
        t = jnp.maximum(x1, 0.0)
        g2 = lax.dot_general(t, w_ref[...], (((1,), (0,)), ((), ())),
                             precision=lax.Precision.HIGHEST,
                             preferred_element_type=jnp.float32)
        o_ref[...] = dis_ref[...] * g2

    return pl.pallas_call(
        body,
        grid=(NPAD // _BR,),
        in_specs=[pl.BlockSpec((NC, _BR, D), lambda i: (0, i, 0)),
                  pl.BlockSpec((_BR, D), lambda i: (i, 0)),
                  pl.BlockSpec((_BR, D), lambda i: (i, 0)),
                  pl.BlockSpec((1, D), lambda i: (0, 0)),
                  pl.BlockSpec((D, D), lambda i: (0, 0))],
        out_specs=pl.BlockSpec((_BR, D), lambda i: (i, 0)),
        out_shape=jax.ShapeDtypeStruct((NPAD, D), jnp.float32),
    )(parts.reshape(NC, NPAD, D), hp1, disf, b1.reshape(1, D), W2)


def _tc_final(parts, hp2, disf, b2):
    def body(p_ref, hp_ref, dis_ref, b_ref, o_ref):
        o = dis_ref[...] * (p_ref[0] + p_ref[1] - hp_ref[...]) + b_ref[...]
        m = jnp.max(o, axis=1, keepdims=True)
        e = jnp.exp(o - m)
        lse = jnp.log(jnp.sum(e, axis=1, keepdims=True))
        o_ref[...] = o - m - lse

    return pl.pallas_call(
        body,
        grid=(NPAD // _BR,),
        in_specs=[pl.BlockSpec((NC, _BR, D), lambda i: (0, i, 0)),
                  pl.BlockSpec((_BR, D), lambda i: (i, 0)),
                  pl.BlockSpec((_BR, D), lambda i: (i, 0)),
                  pl.BlockSpec((1, D), lambda i: (0, 0))],
        out_specs=pl.BlockSpec((_BR, D), lambda i: (i, 0)),
        out_shape=jax.ShapeDtypeStruct((NPAD, D), jnp.float32),
    )(parts.reshape(NC, NPAD, D), hp2, disf, b2.reshape(1, D))


def kernel(x, edge_index, W1, b1, W2, b2):
    ei = edge_index.astype(jnp.int32)
    npe = E_PAD - E
    padv = N + (jnp.arange(npe, dtype=jnp.int32) % N_PADROWS)
    src2d = jnp.concatenate([ei[0], padv]).reshape(EROWS, CHUNK)
    dst2d = jnp.concatenate([ei[1], padv]).reshape(EROWS, CHUNK)
    xp = jnp.pad(x, ((0, NPAD - N), (0, 0)))

    degp = _sc_deg(dst2d)
    hp1, disf = _tc_mm_dis(xp, W1, degp)

    parts1 = _sc_agg(hp1, src2d, dst2d)
    hp2 = _tc_mid(parts1, hp1, disf, b1, W2)

    parts2 = _sc_agg(hp2, src2d, dst2d)
    out = _tc_final(parts2, hp2, disf, b2)
    return out[:N]
